# Initial kernel scaffold; baseline (speedup 1.0000x reference)
import jax
import jax.numpy as jnp
from jax.experimental import pallas as pl

_EMB = 4096
_NE = 64
_K = 8


def _fused_body(x_ref, w_ref, vals_ref, idx_ref):
    x = x_ref[...]
    w = w_ref[...]
    logits = jax.lax.dot_general(
        x, w, (((1,), (1,)), ((), ())),
        preferred_element_type=jnp.float32,
        precision=jax.lax.Precision.HIGHEST)
    iota = jax.lax.broadcasted_iota(jnp.int32, logits.shape, 1)
    work = logits
    vals = []
    idxs = []
    for _ in range(_K):
        m = jnp.max(work, axis=1, keepdims=True)
        amin = jnp.min(jnp.where(work == m, iota, _NE), axis=1, keepdims=True)
        vals.append(m)
        idxs.append(amin)
        work = jnp.where(iota == amin, -jnp.inf, work)
    v = jnp.concatenate(vals, axis=1)
    i = jnp.concatenate(idxs, axis=1)
    e = jnp.exp(v - v[:, :1])
    vals_ref[...] = e / jnp.sum(e, axis=1, keepdims=True)
    idx_ref[...] = i


def kernel(x, W):
    n_tok = x.shape[0]
    bt = 512
    grid = (n_tok // bt,)
    vals, idx = pl.pallas_call(
        _fused_body,
        grid=grid,
        in_specs=[
            pl.BlockSpec((bt, _EMB), lambda i: (i, 0)),
            pl.BlockSpec((_NE, _EMB), lambda i: (0, 0)),
        ],
        out_specs=[
            pl.BlockSpec((bt, _K), lambda i: (i, 0)),
            pl.BlockSpec((bt, _K), lambda i: (i, 0)),
        ],
        out_shape=[
            jax.ShapeDtypeStruct((n_tok, _K), jnp.float32),
            jax.ShapeDtypeStruct((n_tok, _K), jnp.int32),
        ],
    )(x, W)
    return vals, idx


# speedup vs baseline: 1.0609x; 1.0609x over previous
import jax
import jax.numpy as jnp
from jax.experimental import pallas as pl

_EMB = 4096
_NE = 64
_K = 8


def _fused_body(x_ref, w_ref, vals_ref, idx_ref):
    x = x_ref[...]
    w = w_ref[...]
    logits = jax.lax.dot_general(
        x, w, (((1,), (1,)), ((), ())),
        preferred_element_type=jnp.float32,
        precision=jax.lax.Precision.DEFAULT)
    iota = jax.lax.broadcasted_iota(jnp.int32, logits.shape, 1)
    work = logits
    vals = []
    idxs = []
    for _ in range(_K):
        m = jnp.max(work, axis=1, keepdims=True)
        amin = jnp.min(jnp.where(work == m, iota, _NE), axis=1, keepdims=True)
        vals.append(m)
        idxs.append(amin)
        work = jnp.where(iota == amin, -jnp.inf, work)
    v = jnp.concatenate(vals, axis=1)
    i = jnp.concatenate(idxs, axis=1)
    e = jnp.exp(v - v[:, :1])
    vals_ref[...] = e / jnp.sum(e, axis=1, keepdims=True)
    idx_ref[...] = i


def kernel(x, W):
    n_tok = x.shape[0]
    bt = 512
    grid = (n_tok // bt,)
    vals, idx = pl.pallas_call(
        _fused_body,
        grid=grid,
        in_specs=[
            pl.BlockSpec((bt, _EMB), lambda i: (i, 0)),
            pl.BlockSpec((_NE, _EMB), lambda i: (0, 0)),
        ],
        out_specs=[
            pl.BlockSpec((bt, _K), lambda i: (i, 0)),
            pl.BlockSpec((bt, _K), lambda i: (i, 0)),
        ],
        out_shape=[
            jax.ShapeDtypeStruct((n_tok, _K), jnp.float32),
            jax.ShapeDtypeStruct((n_tok, _K), jnp.int32),
        ],
    )(x, W)
    return vals, idx


# speedup vs baseline: 1.1837x; 1.1158x over previous
import jax
import jax.numpy as jnp
from jax.experimental import pallas as pl

_EMB = 4096
_NE = 64
_K = 8


def _fused_body(x_ref, w_ref, vals_ref, idx_ref):
    x = x_ref[...]
    w = w_ref[...]
    logits = jax.lax.dot_general(
        x, w, (((1,), (1,)), ((), ())),
        preferred_element_type=jnp.float32,
        precision=jax.lax.Precision.DEFAULT)
    iota = jax.lax.broadcasted_iota(jnp.int32, logits.shape, 1)
    work = logits
    vals = []
    idxs = []
    for _ in range(_K):
        m = jnp.max(work, axis=1, keepdims=True)
        amin = jnp.min(jnp.where(work == m, iota, _NE), axis=1, keepdims=True)
        vals.append(m)
        idxs.append(amin)
        work = jnp.where(iota == amin, -jnp.inf, work)
    v = jnp.concatenate(vals, axis=1)
    i = jnp.concatenate(idxs, axis=1)
    e = jnp.exp(v - v[:, :1])
    vals_ref[...] = e / jnp.sum(e, axis=1, keepdims=True)
    idx_ref[...] = i


def kernel(x, W):
    n_tok = x.shape[0]
    bt = 1024
    grid = (n_tok // bt,)
    vals, idx = pl.pallas_call(
        _fused_body,
        grid=grid,
        in_specs=[
            pl.BlockSpec((bt, _EMB), lambda i: (i, 0)),
            pl.BlockSpec((_NE, _EMB), lambda i: (0, 0)),
        ],
        out_specs=[
            pl.BlockSpec((bt, _K), lambda i: (i, 0)),
            pl.BlockSpec((bt, _K), lambda i: (i, 0)),
        ],
        out_shape=[
            jax.ShapeDtypeStruct((n_tok, _K), jnp.float32),
            jax.ShapeDtypeStruct((n_tok, _K), jnp.int32),
        ],
    )(x, W)
    return vals, idx
